# baseline (speedup 1.0000x reference)
import functools

import jax
import jax.numpy as jnp
from jax import lax
from jax.experimental import pallas as pl
from jax.experimental.pallas import tpu as pltpu
from jax.experimental.pallas import tpu_sc as plsc

_L = 16


def _entropy_cols(l0, l1):
    m = jnp.maximum(l0, l1)
    e0 = jnp.exp(l0 - m)
    e1 = jnp.exp(l1 - m)
    tot = e0 + e1
    p0 = e0 / tot
    p1 = e1 / tot
    eps = 1e-10
    factor = 1.0 + 0.01 / (1.0 + 1 * 0)
    h = ((p0 + eps) * jnp.log(1.0 / (p0 + eps) + eps)
         + (p1 + eps) * jnp.log(1.0 / (p1 + eps) + eps))
    return h * factor


def _entropy_tc_body(xn_ref, xc_ref, hn_ref, hc_ref):
    hn_ref[...] = _entropy_cols(xn_ref[0, :], xn_ref[1, :])
    hc_ref[...] = _entropy_cols(xc_ref[0, :], xc_ref[1, :])


def _floor16(x):
    t = x.astype(jnp.int32).astype(jnp.float32)
    return t - jnp.where(x < t, 1.0, 0.0)


def _rmw_max(ref, idx, s):
    ks, vi = plsc.sort_key_val(s, idx)
    _, last = plsc.scan_count(vi)
    cur = plsc.load_gather(ref, [vi])
    plsc.store_scatter(ref, [vi], jnp.maximum(cur, ks), mask=last)


def _make_sc_kernel(n_nodes, n_edges):
    try:
        info = plsc.get_sparse_core_info()
        ns = info.num_subcores
    except ValueError:
        ns = 16
    nw = ns
    slc = ((n_nodes + ns * _L - 1) // (ns * _L)) * _L
    n_pad = ns * slc
    chunk = ((n_edges + nw * _L - 1) // (nw * _L)) * _L
    tail = n_edges - (nw - 1) * chunk
    assert tail > 0 and tail % _L == 0 and chunk % 32 == 0
    assert n_pad % 128 == 0 and slc % _L == 0
    mesh = plsc.VectorSubcoreMesh(core_axis_name="c", subcore_axis_name="s",
                                  num_cores=1)

    @functools.partial(
        pl.kernel,
        out_type=(
            jax.ShapeDtypeStruct((n_edges,), jnp.float32),
            jax.ShapeDtypeStruct((ns * n_pad,), jnp.float32),
            jax.ShapeDtypeStruct((n_pad,), jnp.float32),
        ),
        mesh=mesh,
        compiler_params=pltpu.CompilerParams(needs_layout_passes=False),
        scratch_types=[
            pltpu.VMEM((n_pad,), jnp.float32),
            pltpu.VMEM((n_pad,), jnp.float32),
            pltpu.VMEM((chunk,), jnp.int32),
            pltpu.VMEM((chunk,), jnp.int32),
            pltpu.VMEM((chunk,), jnp.float32),
            pltpu.VMEM((chunk,), jnp.float32),
            pltpu.SemaphoreType.DMA,
        ],
    )
    def sc_all(hn, hc, ei, out, partials, global_nm, h_v, nm_v, src_v, dst_v,
               hc_v, sc_v, sem):
        sid = lax.axis_index("s")
        wid = sid
        base = wid * chunk

        h_copy = pltpu.async_copy(hn, h_v.at[pl.ds(0, n_nodes)], sem)

        zeros = jnp.zeros((_L,), jnp.float32)

        def zero_body(j, _):
            for u in range(8):
                nm_v[pl.ds(j * 8 * _L + u * _L, _L)] = zeros
            return 0

        lax.fori_loop(0, n_pad // (8 * _L), zero_body, 0)

        pltpu.sync_copy(ei.at[pl.ds(base, chunk)], src_v)

        @pl.when(wid < nw - 1)
        def _():
            pltpu.sync_copy(ei.at[pl.ds(n_edges + base, chunk)], dst_v)
            pltpu.sync_copy(hc.at[pl.ds(base, chunk)], hc_v)

        @pl.when(wid == nw - 1)
        def _():
            pltpu.sync_copy(ei.at[pl.ds(n_edges + base, tail)],
                            dst_v.at[pl.ds(0, tail)])
            pltpu.sync_copy(hc.at[pl.ds(base, tail)],
                            hc_v.at[pl.ds(0, tail)])
            izeros = jnp.zeros((_L,), jnp.int32)
            for u in range((chunk - tail) // _L):
                dst_v[pl.ds(tail + u * _L, _L)] = izeros
        h_copy.wait()

        iota = lax.iota(jnp.int32, _L)

        def edge_body(j, _):
            for u in range(2):
                off = (j * 2 + u) * _L
                sl = pl.ds(off, _L)
                si = src_v[sl]
                di = dst_v[sl]
                hcv = hc_v[sl]
                hs = plsc.load_gather(h_v, [si])
                hd = plsc.load_gather(h_v, [di])
                a = hs - hcv
                b = hd - hcv
                fa = _floor16(a)
                fb = _floor16(b)
                s = (2.0 + a) * (2.0 + b) * ((1.0 + fa) * (1.0 + fb))
                lane = base + off + iota
                s = jnp.where(lane < n_edges, s, 0.0)
                sc_v[sl] = s
                _rmw_max(nm_v, si, s)
                _rmw_max(nm_v, di, s)
            return 0

        lax.fori_loop(0, chunk // (2 * _L), edge_body, 0)

        pltpu.sync_copy(nm_v, partials.at[pl.ds(sid * n_pad, n_pad)])
        plsc.subcore_barrier()
        red_copy = None
        for t in range(ns):
            red_copy = pltpu.async_copy(
                partials.at[pl.ds(t * n_pad + sid * slc, slc)],
                h_v.at[pl.ds(t * slc, slc)], sem)
        for t in range(ns):
            red_copy.wait()

        def red_body(j, _):
            off = j * _L
            acc = h_v[pl.ds(off, _L)]
            for t in range(1, ns):
                acc = jnp.maximum(acc, h_v[pl.ds(t * slc + off, _L)])
            nm_v[pl.ds(off, _L)] = acc
            return 0

        lax.fori_loop(0, slc // _L, red_body, 0)
        pltpu.sync_copy(nm_v.at[pl.ds(0, slc)],
                        global_nm.at[pl.ds(sid * slc, slc)])
        plsc.subcore_barrier()
        pltpu.sync_copy(global_nm, h_v.at[pl.ds(0, n_pad)])

        def sel_body(j, _):
            for u in range(2):
                sl = pl.ds((j * 2 + u) * _L, _L)
                s = sc_v[sl]
                ms = plsc.load_gather(h_v, [src_v[sl]])
                md = plsc.load_gather(h_v, [dst_v[sl]])
                keep = (s > 0.0) & (s >= ms) & (s >= md)
                sc_v[sl] = jnp.where(keep, s, 0.0)
            return 0

        lax.fori_loop(0, chunk // (2 * _L), sel_body, 0)

        @pl.when(wid < nw - 1)
        def _():
            pltpu.sync_copy(sc_v, out.at[pl.ds(base, chunk)])

        @pl.when(wid == nw - 1)
        def _():
            pltpu.sync_copy(sc_v.at[pl.ds(0, tail)], out.at[pl.ds(base, tail)])

    return sc_all


@jax.jit
def kernel(node_logits, comb_logits, edge_index):
    n_nodes = node_logits.shape[0]
    n_edges = comb_logits.shape[0]
    sc_all = _make_sc_kernel(n_nodes, n_edges)

    xn = node_logits.T
    xc = comb_logits.T
    hn, hc = pl.pallas_call(
        _entropy_tc_body,
        out_shape=(
            jax.ShapeDtypeStruct((n_nodes,), jnp.float32),
            jax.ShapeDtypeStruct((n_edges,), jnp.float32),
        ),
    )(xn, xc)
    ei = edge_index.reshape(2 * n_edges)
    return sc_all(hn, hc, ei)[0]
